# Initial kernel scaffold; baseline (speedup 1.0000x reference)
import jax
import jax.numpy as jnp
from jax.experimental import pallas as pl


def kernel(tokenized_text, token_embedding, ctx, token_prefix):
    raise NotImplementedError("write your pallas kernel here")


# speedup vs baseline: 1.1128x; 1.1128x over previous
import functools

import jax
import jax.numpy as jnp
from jax import lax
from jax.experimental import pallas as pl
from jax.experimental.pallas import tpu as pltpu
from jax.experimental.pallas import tpu_sc as plsc

N_CLS = 1000
N_CTX = 16
D = 512
SEQ = 77
TOK = SEQ - 1
ROWS = 1 + N_CTX + TOK
NC, NS = 2, 16
NW = NC * NS
CPW = 32


@functools.partial(
    pl.kernel,
    out_type=jax.ShapeDtypeStruct((N_CLS * ROWS, D), jnp.float32),
    mesh=plsc.VectorSubcoreMesh(
        core_axis_name="c", subcore_axis_name="s",
        num_cores=NC, num_subcores=NS,
    ),
    scratch_types=[
        pltpu.VMEM((CPW, TOK), jnp.int32),
        pltpu.VMEM((ROWS, D), jnp.float32),
        pltpu.SemaphoreType.DMA,
    ],
    compiler_params=pltpu.CompilerParams(use_tc_tiling_on_sc=False),
)
def _prompt_assemble(idx_hbm, emb_hbm, ctx_hbm, pref_hbm, out_hbm,
                     idx_v, buf, sem):
    wid = lax.axis_index("s") * NC + lax.axis_index("c")
    base = jnp.minimum(wid * CPW, N_CLS - CPW)
    pltpu.sync_copy(idx_hbm.at[pl.ds(base, CPW)], idx_v)
    pltpu.sync_copy(ctx_hbm, buf.at[pl.ds(1, N_CTX)])

    def step(t, carry):
        c = base + t
        pltpu.sync_copy(pref_hbm.at[pl.ds(c, 1)], buf.at[pl.ds(0, 1)])
        pltpu.async_copy(emb_hbm.at[idx_v.at[t]],
                         buf.at[pl.ds(1 + N_CTX, TOK)], sem).wait()
        pltpu.sync_copy(buf, out_hbm.at[pl.ds(c * ROWS, ROWS)])
        return carry

    lax.fori_loop(0, CPW, step, 0)


def kernel(tokenized_text, token_embedding, ctx, token_prefix):
    idx = tokenized_text[:, 1:].astype(jnp.int32)
    ctx0 = ctx[0]
    pref = token_prefix.reshape(N_CLS, D)
    out = _prompt_assemble(idx, token_embedding, ctx0, pref)
    return out.reshape(N_CLS, ROWS, D)


# speedup vs baseline: 1.1486x; 1.0322x over previous
import functools

import jax
import jax.numpy as jnp
from jax import lax
from jax.experimental import pallas as pl
from jax.experimental.pallas import tpu as pltpu
from jax.experimental.pallas import tpu_sc as plsc

N_CLS = 1000
N_CTX = 16
D = 512
SEQ = 77
TOK = SEQ - 1
HDR = 1 + N_CTX
ROWS = HDR + TOK
NC, NS = 2, 16
NW = NC * NS
CPW = 32


@functools.partial(
    pl.kernel,
    out_type=jax.ShapeDtypeStruct((N_CLS * ROWS, D), jnp.float32),
    mesh=plsc.VectorSubcoreMesh(
        core_axis_name="c", subcore_axis_name="s",
        num_cores=NC, num_subcores=NS,
    ),
    scratch_types=[
        pltpu.VMEM((CPW, TOK), jnp.int32),
        pltpu.VMEM((ROWS, D), jnp.float32),
        pltpu.VMEM((ROWS, D), jnp.float32),
        pltpu.SemaphoreType.DMA,
        pltpu.SemaphoreType.DMA,
        pltpu.SemaphoreType.DMA,
        pltpu.SemaphoreType.DMA,
    ],
    compiler_params=pltpu.CompilerParams(use_tc_tiling_on_sc=False),
)
def _prompt_assemble(idx_hbm, emb_hbm, ctx_hbm, pref_hbm, out_hbm,
                     idx_v, buf_a, buf_b,
                     sem_ain, sem_aout, sem_bin, sem_bout):
    wid = lax.axis_index("s") * NC + lax.axis_index("c")
    base = jnp.minimum(wid * CPW, N_CLS - CPW)

    bufs = ((buf_a, sem_ain, sem_aout), (buf_b, sem_bin, sem_bout))

    def gather_start(buf, t, sem):
        pltpu.make_async_copy(pref_hbm.at[pl.ds(base + t, 1)],
                              buf.at[pl.ds(0, 1)], sem).start()
        pltpu.make_async_copy(emb_hbm.at[idx_v.at[t]],
                              buf.at[pl.ds(HDR, TOK)], sem).start()

    def gather_wait(buf, sem):
        pltpu.make_async_copy(pref_hbm.at[pl.ds(0, 1)],
                              buf.at[pl.ds(0, 1)], sem).wait()
        pltpu.make_async_copy(emb_hbm.at[idx_v.at[0]],
                              buf.at[pl.ds(HDR, TOK)], sem).wait()

    def scatter_start(buf, t, sem):
        pltpu.make_async_copy(buf, out_hbm.at[pl.ds((base + t) * ROWS, ROWS)],
                              sem).start()

    def scatter_wait(buf, sem):
        pltpu.make_async_copy(buf, out_hbm.at[pl.ds(0, ROWS)], sem).wait()

    pltpu.sync_copy(idx_hbm.at[pl.ds(base, CPW)], idx_v)
    pltpu.sync_copy(ctx_hbm, buf_a.at[pl.ds(1, N_CTX)])
    pltpu.sync_copy(ctx_hbm, buf_b.at[pl.ds(1, N_CTX)])
    for k, (buf, sin, _) in enumerate(bufs):
        gather_start(buf, k, sin)

    def body(i, carry):
        for k, (buf, sin, sout) in enumerate(bufs):
            gather_wait(buf, sin)
            scatter_start(buf, 2 * i + k, sout)
        for k, (buf, sin, sout) in enumerate(bufs):
            scatter_wait(buf, sout)
            gather_start(buf, 2 * i + k + 2, sin)
        return carry

    lax.fori_loop(0, CPW // 2 - 1, body, 0)

    for k, (buf, sin, sout) in enumerate(bufs):
        gather_wait(buf, sin)
        scatter_start(buf, CPW - 2 + k, sout)
    for _, (buf, _, sout) in enumerate(bufs):
        scatter_wait(buf, sout)


def kernel(tokenized_text, token_embedding, ctx, token_prefix):
    idx = tokenized_text[:, 1:].astype(jnp.int32)
    ctx0 = ctx[0]
    pref = token_prefix.reshape(N_CLS, D)
    out = _prompt_assemble(idx, token_embedding, ctx0, pref)
    return out.reshape(N_CLS, ROWS, D)
